# baseline (speedup 1.0000x reference)
import functools

import jax
import jax.numpy as jnp
from jax import lax
from jax.experimental import pallas as pl
from jax.experimental.pallas import tpu as pltpu
from jax.experimental.pallas import tpu_sc as plsc

_NC = 2
_NS = 16
_NW = _NC * _NS
_CHUNK = 128

_FUSE_CB = 12800


def _fuse_body(scale, embt_ref, at_ref, bw_ref, bias_ref, out_ref):
    acc = jnp.dot(bw_ref[...], at_ref[...], preferred_element_type=jnp.float32)
    ft = embt_ref[...] + scale * acc + scale * bias_ref[...]
    out_ref[...] = jnp.swapaxes(ft, 0, 1)


def _fused_table(embt, at, bw, bias2d, scale, num_emb, emb_dim):
    grid = ((num_emb + _FUSE_CB - 1) // _FUSE_CB,)
    hidden = at.shape[0]
    out2 = pl.pallas_call(
        functools.partial(_fuse_body, scale),
        grid=grid,
        in_specs=[
            pl.BlockSpec((emb_dim, _FUSE_CB), lambda i: (0, i)),
            pl.BlockSpec((hidden, _FUSE_CB), lambda i: (0, i)),
            pl.BlockSpec((emb_dim, hidden), lambda i: (0, 0)),
            pl.BlockSpec((emb_dim, 1), lambda i: (0, 0)),
        ],
        out_specs=pl.BlockSpec((_FUSE_CB, emb_dim), lambda i: (i, 0)),
        out_shape=jax.ShapeDtypeStruct((num_emb, emb_dim), jnp.float32),
    )(embt, at, bw, bias2d)
    return out2


@functools.lru_cache(maxsize=None)
def _make_gather(n_tok, emb_dim):
    bpw = n_tok // _NW
    nch = bpw // _CHUNK
    mesh = plsc.VectorSubcoreMesh(core_axis_name="c", subcore_axis_name="s")

    @functools.partial(
        pl.kernel,
        mesh=mesh,
        compiler_params=pltpu.CompilerParams(use_tc_tiling_on_sc=False),
        out_type=jax.ShapeDtypeStruct((n_tok, emb_dim), jnp.float32),
        scratch_types=[
            pltpu.VMEM((nch, _CHUNK), jnp.int32),
            pltpu.VMEM((_CHUNK, emb_dim), jnp.float32),
            pltpu.SemaphoreType.DMA,
        ],
    )
    def gather_kernel(table_hbm, idx_hbm, out_hbm, idx_v, rows0, sem0):
        wid = lax.axis_index("s") * _NC + lax.axis_index("c")
        base = wid * bpw
        pltpu.sync_copy(idx_hbm.at[wid], idx_v)

        def body(j, _):
            cp = pltpu.async_copy(table_hbm.at[idx_v.at[j]], rows0, sem0)
            cp.wait()
            pltpu.sync_copy(rows0, out_hbm.at[pl.ds(base + j * _CHUNK, _CHUNK)])
            return 0

        lax.fori_loop(0, nch, body, 0, unroll=False)

    return gather_kernel


def _untrans_body(g3_ref, out_ref):
    xt = jnp.swapaxes(g3_ref[0], 0, 1)
    j = pl.program_id(1)
    nb = g3_ref.shape[1]
    half = out_ref.shape[2] // 2
    d = out_ref.shape[1]
    out_ref[0, :, pl.ds(j * nb, nb)] = xt[0:d, :]
    out_ref[0, :, pl.ds(half + j * nb, nb)] = xt[d:2 * d, :]


def _untranspose(g3, seq_len, batch, emb_dim):
    nb = 256
    grid = (seq_len, batch // 2 // nb)
    return pl.pallas_call(
        _untrans_body,
        grid=grid,
        in_specs=[pl.BlockSpec((1, nb, 2 * emb_dim), lambda l, j: (l, j, 0))],
        out_specs=pl.BlockSpec((1, emb_dim, batch), lambda l, j: (l, 0, 0)),
        out_shape=jax.ShapeDtypeStruct((seq_len, emb_dim, batch), jnp.float32),
    )(g3)


def kernel(x, emb_W, A_W, B_W, B_b):
    num_emb, emb_dim = emb_W.shape
    hidden = A_W.shape[1]
    batch, seq_len = x.shape
    scale = 0.1 / hidden

    table = _fused_table(emb_W.T, A_W.T, B_W, B_b.reshape(emb_dim, 1),
                         scale, num_emb, emb_dim)

    xt = x.T
    idx3 = xt.reshape(seq_len, 2, batch // 2).transpose(0, 2, 1)
    n_tok = batch * seq_len
    idx = idx3.reshape(_NW, n_tok // _NW // _CHUNK, _CHUNK)

    g = _make_gather(n_tok, emb_dim)(table, idx)
    g3 = g.reshape(seq_len, batch // 2, 2 * emb_dim)
    t3 = _untranspose(g3, seq_len, batch, emb_dim)
    return t3.transpose(2, 0, 1)


# speedup vs baseline: 1.1329x; 1.1329x over previous
import functools

import jax
import jax.numpy as jnp
from jax import lax
from jax.experimental import pallas as pl
from jax.experimental.pallas import tpu as pltpu
from jax.experimental.pallas import tpu_sc as plsc

_NC = 2
_NS = 16
_NW = _NC * _NS
_LANES = 16

_FUSE_CB = 12800


def _fuse_body(scale, embt_ref, at_ref, bw_ref, bias_ref, out_ref):
    acc = jnp.dot(bw_ref[...], at_ref[...], preferred_element_type=jnp.float32)
    out_ref[...] = embt_ref[...] + scale * acc + scale * bias_ref[...]


def _fused_table_t(embt, at, bw, bias2d, scale, num_emb, emb_dim):
    nblk = (num_emb + _FUSE_CB - 1) // _FUSE_CB
    hidden = at.shape[0]
    return pl.pallas_call(
        functools.partial(_fuse_body, scale),
        grid=(nblk,),
        in_specs=[
            pl.BlockSpec((emb_dim, _FUSE_CB), lambda i: (0, i)),
            pl.BlockSpec((hidden, _FUSE_CB), lambda i: (0, i)),
            pl.BlockSpec((emb_dim, hidden), lambda i: (0, 0)),
            pl.BlockSpec((emb_dim, 1), lambda i: (0, 0)),
        ],
        out_specs=pl.BlockSpec((emb_dim, _FUSE_CB), lambda i: (0, i)),
        out_shape=jax.ShapeDtypeStruct((emb_dim, num_emb), jnp.float32),
    )(embt, at, bw, bias2d)


@functools.lru_cache(maxsize=None)
def _make_permute(seq_len, batch, emb_dim, num_emb):
    d_per_w = emb_dim // _NW
    n_vec = batch // _LANES
    mesh = plsc.VectorSubcoreMesh(core_axis_name="c", subcore_axis_name="s")

    @functools.partial(
        pl.kernel,
        mesh=mesh,
        compiler_params=pltpu.CompilerParams(
            use_tc_tiling_on_sc=True, needs_layout_passes=False),
        out_type=jax.ShapeDtypeStruct((seq_len, emb_dim, batch), jnp.float32),
        scratch_types=[
            pltpu.VMEM((num_emb,), jnp.float32),
            pltpu.VMEM((batch,), jnp.int32),
            pltpu.VMEM((batch,), jnp.float32),
        ],
    )
    def permute_kernel(ft_hbm, xt_hbm, out_hbm, row_v, idx_v, stage_v):
        wid = lax.axis_index("s") * _NC + lax.axis_index("c")

        for dd in range(d_per_w):
            d = wid * d_per_w + dd
            pltpu.sync_copy(ft_hbm.at[d], row_v)

            @pl.loop(0, seq_len)
            def _per_l(l, d=d):
                pltpu.sync_copy(xt_hbm.at[l], idx_v)

                @pl.loop(0, n_vec, unroll=8)
                def _per_vec(k):
                    iv = idx_v[pl.ds(k * _LANES, _LANES)]
                    stage_v[pl.ds(k * _LANES, _LANES)] = (
                        plsc.load_gather(row_v, [iv]))

                pltpu.sync_copy(stage_v, out_hbm.at[l, d])

    return permute_kernel


def kernel(x, emb_W, A_W, B_W, B_b):
    num_emb, emb_dim = emb_W.shape
    hidden = A_W.shape[1]
    batch, seq_len = x.shape
    scale = 0.1 / hidden

    ft = _fused_table_t(emb_W.T, A_W.T, B_W, B_b.reshape(emb_dim, 1),
                        scale, num_emb, emb_dim)
    t3 = _make_permute(seq_len, batch, emb_dim, ft.shape[1])(ft, x.T)
    return t3.transpose(2, 0, 1)


# speedup vs baseline: 1.5348x; 1.3548x over previous
import functools

import jax
import jax.numpy as jnp
from jax import lax
from jax.experimental import pallas as pl
from jax.experimental.pallas import tpu as pltpu
from jax.experimental.pallas import tpu_sc as plsc

_NC = 2
_NS = 16
_NW = _NC * _NS
_LANES = 16

_FUSE_CB = 12800


def _fuse_body(scale, embt_ref, at_ref, bw_ref, bias_ref, out_ref):
    acc = jnp.dot(bw_ref[...], at_ref[...], preferred_element_type=jnp.float32)
    out_ref[...] = embt_ref[...] + scale * acc + scale * bias_ref[...]


def _fused_table_t(embt, at, bw, bias2d, scale, num_emb, emb_dim):
    nblk = (num_emb + _FUSE_CB - 1) // _FUSE_CB
    hidden = at.shape[0]
    return pl.pallas_call(
        functools.partial(_fuse_body, scale),
        grid=(nblk,),
        in_specs=[
            pl.BlockSpec((emb_dim, _FUSE_CB), lambda i: (0, i)),
            pl.BlockSpec((hidden, _FUSE_CB), lambda i: (0, i)),
            pl.BlockSpec((emb_dim, hidden), lambda i: (0, 0)),
            pl.BlockSpec((emb_dim, 1), lambda i: (0, 0)),
        ],
        out_specs=pl.BlockSpec((emb_dim, _FUSE_CB), lambda i: (0, i)),
        out_shape=jax.ShapeDtypeStruct((emb_dim, num_emb), jnp.float32),
    )(embt, at, bw, bias2d)


_NB = 3


@functools.lru_cache(maxsize=None)
def _make_permute(seq_len, batch, emb_dim, num_emb):
    d_per_w = emb_dim // _NW
    n_vec = batch // _LANES
    stage_bytes = batch * 4
    n_main = (seq_len // _NB) - 1
    mesh = plsc.VectorSubcoreMesh(core_axis_name="c", subcore_axis_name="s")

    @functools.partial(
        pl.kernel,
        mesh=mesh,
        compiler_params=pltpu.CompilerParams(
            use_tc_tiling_on_sc=True, needs_layout_passes=False),
        out_type=jax.ShapeDtypeStruct((seq_len, emb_dim, batch), jnp.float32),
        scratch_types=(
            [pltpu.VMEM((num_emb,), jnp.float32)]
            + [pltpu.VMEM((batch,), jnp.int32)] * _NB
            + [pltpu.VMEM((batch,), jnp.float32)] * _NB
            + [pltpu.SemaphoreType.DMA] * (2 * _NB)
        ),
    )
    def permute_kernel(ft_hbm, xt_hbm, out_hbm, row_v, *rest):
        ibufs = rest[:_NB]
        sbufs = rest[_NB:2 * _NB]
        isems = rest[2 * _NB:3 * _NB]
        wsems = rest[3 * _NB:]
        wid = lax.axis_index("s") * _NC + lax.axis_index("c")

        def step(p, l, d, prefetch, first=False):
            pltpu.make_async_copy(xt_hbm.at[0], ibufs[p], isems[p]).wait()
            if not first:
                pltpu.make_async_copy(sbufs[p], out_hbm.at[0, 0],
                                      wsems[p]).wait()

            @pl.loop(0, n_vec, unroll=8)
            def _per_vec(k):
                iv = ibufs[p][pl.ds(k * _LANES, _LANES)]
                sbufs[p][pl.ds(k * _LANES, _LANES)] = (
                    plsc.load_gather(row_v, [iv]))

            pltpu.async_copy(sbufs[p], out_hbm.at[l, d], wsems[p])
            if prefetch:
                pltpu.async_copy(xt_hbm.at[l + _NB], ibufs[p], isems[p])

        for dd in range(d_per_w):
            d = wid * d_per_w + dd
            pltpu.sync_copy(ft_hbm.at[d], row_v)
            for p in range(_NB):
                pltpu.async_copy(xt_hbm.at[p], ibufs[p], isems[p])

            for p in range(_NB):
                step(p, p, d, prefetch=True, first=True)

            @pl.loop(1, n_main)
            def _main(lp, d=d):
                for p in range(_NB):
                    step(p, lp * _NB + p, d, prefetch=True)

            for l in range(n_main * _NB, seq_len):
                step(l % _NB, l, d, prefetch=(l + _NB < seq_len))

            for p in range(_NB):
                pltpu.make_async_copy(
                    sbufs[p], out_hbm.at[0, 0], wsems[p]).wait()

    return permute_kernel


def kernel(x, emb_W, A_W, B_W, B_b):
    num_emb, emb_dim = emb_W.shape
    hidden = A_W.shape[1]
    batch, seq_len = x.shape
    scale = 0.1 / hidden

    ft = _fused_table_t(emb_W.T, A_W.T, B_W, B_b.reshape(emb_dim, 1),
                        scale, num_emb, emb_dim)
    t3 = _make_permute(seq_len, batch, emb_dim, ft.shape[1])(ft, x.T)
    return t3.transpose(2, 0, 1)
